# baseline (speedup 1.0000x reference)
import dataclasses
import functools

import jax
import jax.numpy as jnp
from jax import lax
from jax.experimental import pallas as pl
from jax.experimental.pallas import tpu as pltpu
from jax.experimental.pallas import tpu_sc as plsc

NC = 2
NS = 16
LN = 16
CH = 112

_PERM32 = [0, 16, 1, 17, 2, 18, 3, 19, 4, 20, 5, 21, 6, 22, 7, 23,
           8, 24, 9, 25, 10, 26, 11, 27, 12, 28, 13, 29, 14, 30, 15, 31]


def _lane_splat(vec, i):
    idx = jnp.full((LN, 1), i, jnp.int32)
    dn = lax.GatherDimensionNumbers(
        offset_dims=(), collapsed_slice_dims=(0,), start_index_map=(0,))
    return lax.gather(vec, idx, dn, slice_sizes=(1,),
                      mode=lax.GatherScatterMode.PROMISE_IN_BOUNDS)


def _scale_y_body(y_ref, deg_ref, lam_ref, yp_ref):
    lam = lam_ref[0, 0]
    dl = lam * deg_ref[...] + (1.0 - lam)
    yp_ref[...] = (y_ref[...] * lax.rsqrt(dl)).astype(jnp.bfloat16)


def _combine_body(y_ref, x_ref, deg_ref, h_ref, alp_ref, lam_ref, o_ref):
    alp = alp_ref[0, 0]
    lam = lam_ref[0, 0]
    dl = lam * deg_ref[...] + (1.0 - lam)
    nh = lax.rsqrt(dl)
    agg = h_ref[0] + h_ref[1]
    o_ref[...] = ((1.0 - alp) * y_ref[...]
                  + (alp * lam) * (nh * agg)
                  + alp * (x_ref[...] / dl))


def _make_sc_kernel(n2, d, chunks):
    rows_per_tile = n2 // NS
    mesh = plsc.VectorSubcoreMesh(core_axis_name="c", subcore_axis_name="s")
    cp = pltpu.CompilerParams()
    for field, val in (("needs_layout_passes", False),
                       ("use_tc_tiling_on_sc", False)):
        if field in pltpu.CompilerParams.__dataclass_fields__:
            cp = dataclasses.replace(cp, **{field: val})

    @functools.partial(
        pl.kernel,
        mesh=mesh,
        compiler_params=cp,
        out_type=jax.ShapeDtypeStruct((NC, n2, d), jnp.float32),
        scratch_types=[
            pltpu.VMEM((4, CH), jnp.int32),
            pltpu.VMEM((4, CH), jnp.int32),
            pltpu.VMEM((4, CH), jnp.float32),
            pltpu.VMEM((2 * CH, d), jnp.bfloat16),
            pltpu.VMEM((2 * CH, d), jnp.float32),
            pltpu.SemaphoreType.DMA,
            pltpu.SemaphoreType.DMA,
            pltpu.SemaphoreType.DMA,
            pltpu.SemaphoreType.DMA,
            pltpu.SemaphoreType.DMA,
            pltpu.SemaphoreType.DMA,
            pltpu.SemaphoreType.DMA,
            pltpu.SemaphoreType.DMA,
            pltpu.VMEM_SHARED((n2, d), jnp.float32),
        ],
    )
    def sc_fn(ybf, srcs, dsts, ws, out,
              src_r, dst_r, w_r, grows_v, frows_v,
              e0, e1, e2, e3, g0, g1, s0, s1, acc):
        esem = (e0, e1, e2, e3)
        gsem = (g0, g1)
        ssem = (s0, s1)
        c = lax.axis_index("c")
        s = lax.axis_index("s")
        base = s * rows_per_tile

        @pl.loop(0, 2 * CH)
        def _zero_row(r):
            for j in range(d // LN):
                frows_v[r, pl.ds(j * LN, LN)] = jnp.zeros((LN,), jnp.float32)

        for m in (2, 3):
            for j in range(CH // LN):
                dst_r[m, pl.ds(j * LN, LN)] = jnp.zeros((LN,), jnp.int32)

        n_full, rem = divmod(rows_per_tile, CH)
        for k in range(n_full):
            pltpu.sync_copy(frows_v.at[pl.ds(0, CH)],
                            acc.at[pl.ds(base + k * CH, CH)])
        if rem:
            pltpu.sync_copy(frows_v.at[pl.ds(0, rem)],
                            acc.at[pl.ds(base + n_full * CH, rem)])

        plsc.subcore_barrier()

        def gbuf(b):
            return grows_v.at[pl.ds(b * CH, CH)]

        def fbuf(b):
            return frows_v.at[pl.ds(b * CH, CH)]

        def e_start(m, ci):
            pltpu.async_copy(srcs.at[c, s, ci], src_r.at[m], esem[m])
            pltpu.async_copy(dsts.at[c, s, ci], dst_r.at[m], esem[m])
            pltpu.async_copy(ws.at[c, s, ci], w_r.at[m], esem[m])

        def e_wait(m, ci):
            pltpu.make_async_copy(srcs.at[c, s, ci], src_r.at[m],
                                  esem[m]).wait()
            pltpu.make_async_copy(dsts.at[c, s, ci], dst_r.at[m],
                                  esem[m]).wait()
            pltpu.make_async_copy(ws.at[c, s, ci], w_r.at[m],
                                  esem[m]).wait()

        def g_start(b, m):
            pltpu.async_copy(ybf.at[src_r.at[m]], gbuf(b), gsem[b])

        def g_wait(b, m):
            pltpu.make_async_copy(ybf.at[src_r.at[m]], gbuf(b),
                                  gsem[b]).wait()

        def s_start(b, m):
            pltpu.async_copy(fbuf(b), acc.at[dst_r.at[m]], ssem[b], add=True)

        def s_wait(b, m):
            pltpu.make_async_copy(fbuf(b), acc.at[dst_r.at[m]],
                                  ssem[b]).wait()

        def scale(b, m):
            @pl.loop(0, CH // LN)
            def _grp(g):
                wv = w_r[m, pl.ds(g * LN, LN)]
                for i in range(LN):
                    sp = _lane_splat(wv, i)
                    r = b * CH + g * LN + i
                    for g2 in range(d // 32):
                        packed = grows_v[r, pl.ds(g2 * 32, 32)]
                        lo, hi = plsc.unpack(
                            packed, format=plsc.PackFormat.INTERLEAVED)
                        frows_v[r, pl.ds(g2 * 32, LN)] = lo * sp
                        frows_v[r, pl.ds(g2 * 32 + LN, LN)] = hi * sp

        s_start(0, 2)
        s_start(1, 3)
        e_start(0, 0)
        e_start(1, 1)
        e_wait(0, 0)
        g_start(0, 0)

        n_iter = chunks // 4

        @pl.loop(0, n_iter)
        def _ring(h):
            c0 = 4 * h
            for k in range(4):
                ci = c0 + k
                b = k % 2
                mn = (k + 1) % 4
                s_wait(b, k)
                if k < 2:
                    e_start(k + 2, ci + 2)
                else:
                    @pl.when(h < n_iter - 1)
                    def _epf():
                        e_start((k + 2) % 4, ci + 2)
                if k < 3:
                    e_wait(mn, ci + 1)
                    g_start(1 - b, mn)
                else:
                    @pl.when(h < n_iter - 1)
                    def _gpf():
                        e_wait(mn, ci + 1)
                        g_start(1 - b, mn)
                g_wait(b, k)
                scale(b, k)
                s_start(b, k)

        s_wait(0, 2)
        s_wait(1, 3)

        plsc.subcore_barrier()
        pltpu.sync_copy(acc.at[pl.ds(base, rows_per_tile)],
                        out.at[c, pl.ds(base, rows_per_tile)])

    return sc_fn


def kernel(Y, X, edge_weight, deg, alp, lam, edge_index):
    n, d = Y.shape
    e = edge_weight.shape[0]
    chunks = 4 * (-(-e // (NC * NS * CH * 4)))
    epad = NC * NS * chunks * CH
    n2 = NS * 8 * (-(-n // (NS * 8)))

    src = edge_index[0].astype(jnp.int32)
    dst = edge_index[1].astype(jnp.int32)
    w = edge_weight.astype(jnp.float32)
    pad = epad - e
    if pad:
        src = jnp.concatenate([src, jnp.zeros((pad,), jnp.int32)])
        dst = jnp.concatenate([dst, jnp.zeros((pad,), jnp.int32)])
        w = jnp.concatenate([w, jnp.zeros((pad,), jnp.float32)])
    src4 = src.reshape(NC, NS, chunks, CH)
    dst4 = dst.reshape(NC, NS, chunks, CH)
    w4 = w.reshape(NC, NS, chunks, CH)
    ypad = Y
    deg_pad = deg
    if n2 > n:
        ypad = jnp.concatenate([Y, jnp.zeros((n2 - n, d), jnp.float32)])
        deg_pad = jnp.concatenate([deg, jnp.ones((n2 - n,), jnp.float32)])
    lam11 = lam.reshape(1, 1)
    alp11 = alp.reshape(1, 1)

    ybf = pl.pallas_call(
        _scale_y_body,
        out_shape=jax.ShapeDtypeStruct((n2, d), jnp.bfloat16),
    )(ypad, deg_pad[:, None], lam11)
    perm = jnp.asarray([g * 32 + p for g in range(d // 32) for p in _PERM32],
                       dtype=jnp.int32)
    ybf = ybf[:, perm]

    partials = _make_sc_kernel(n2, d, chunks)(ybf, src4, dst4, w4)[:, :n, :]

    blk = 2000
    out = pl.pallas_call(
        _combine_body,
        grid=(n // blk,),
        in_specs=[
            pl.BlockSpec((blk, d), lambda i: (i, 0)),
            pl.BlockSpec((blk, d), lambda i: (i, 0)),
            pl.BlockSpec((blk, 1), lambda i: (i, 0)),
            pl.BlockSpec((NC, blk, d), lambda i: (0, i, 0)),
            pl.BlockSpec((1, 1), lambda i: (0, 0)),
            pl.BlockSpec((1, 1), lambda i: (0, 0)),
        ],
        out_specs=pl.BlockSpec((blk, d), lambda i: (i, 0)),
        out_shape=jax.ShapeDtypeStruct((n, d), jnp.float32),
    )(Y, X, deg[:, None], partials, alp11, lam11)
    return out


# speedup vs baseline: 1.0246x; 1.0246x over previous
import dataclasses
import functools

import jax
import jax.numpy as jnp
from jax import lax
from jax.experimental import pallas as pl
from jax.experimental.pallas import tpu as pltpu
from jax.experimental.pallas import tpu_sc as plsc

NC = 2
NS = 16
LN = 16
CH = 112

_PERM32 = [0, 16, 1, 17, 2, 18, 3, 19, 4, 20, 5, 21, 6, 22, 7, 23,
           8, 24, 9, 25, 10, 26, 11, 27, 12, 28, 13, 29, 14, 30, 15, 31]


def _lane_splat(vec, i):
    idx = jnp.full((LN, 1), i, jnp.int32)
    dn = lax.GatherDimensionNumbers(
        offset_dims=(), collapsed_slice_dims=(0,), start_index_map=(0,))
    return lax.gather(vec, idx, dn, slice_sizes=(1,),
                      mode=lax.GatherScatterMode.PROMISE_IN_BOUNDS)


def _scale_y_body(y_ref, deg_ref, lam_ref, yp_ref):
    lam = lam_ref[0, 0]
    dl = lam * deg_ref[...] + (1.0 - lam)
    yp_ref[...] = (y_ref[...] * lax.rsqrt(dl)).astype(jnp.bfloat16)


def _combine_body(y_ref, x_ref, deg_ref, h_ref, alp_ref, lam_ref, o_ref):
    alp = alp_ref[0, 0]
    lam = lam_ref[0, 0]
    dl = lam * deg_ref[...] + (1.0 - lam)
    nh = lax.rsqrt(dl)
    agg = (h_ref[0].astype(jnp.float32)
           + h_ref[1].astype(jnp.float32))
    o_ref[...] = ((1.0 - alp) * y_ref[...]
                  + (alp * lam) * (nh * agg)
                  + alp * (x_ref[...] / dl))


def _make_sc_kernel(n2, d, chunks):
    rows_per_tile = n2 // NS
    mesh = plsc.VectorSubcoreMesh(core_axis_name="c", subcore_axis_name="s")
    cp = pltpu.CompilerParams()
    for field, val in (("needs_layout_passes", False),
                       ("use_tc_tiling_on_sc", False)):
        if field in pltpu.CompilerParams.__dataclass_fields__:
            cp = dataclasses.replace(cp, **{field: val})

    @functools.partial(
        pl.kernel,
        mesh=mesh,
        compiler_params=cp,
        out_type=jax.ShapeDtypeStruct((NC, n2, d), jnp.bfloat16),
        scratch_types=[
            pltpu.VMEM((4, CH), jnp.int32),
            pltpu.VMEM((4, CH), jnp.int32),
            pltpu.VMEM((4, CH), jnp.float32),
            pltpu.VMEM((2 * CH, d), jnp.bfloat16),
            pltpu.VMEM((2 * CH, d), jnp.bfloat16),
            pltpu.SemaphoreType.DMA,
            pltpu.SemaphoreType.DMA,
            pltpu.SemaphoreType.DMA,
            pltpu.SemaphoreType.DMA,
            pltpu.SemaphoreType.DMA,
            pltpu.SemaphoreType.DMA,
            pltpu.SemaphoreType.DMA,
            pltpu.SemaphoreType.DMA,
            pltpu.VMEM_SHARED((n2, d), jnp.bfloat16),
        ],
    )
    def sc_fn(ybf, srcs, dsts, ws, out,
              src_r, dst_r, w_r, grows_v, frows_v,
              e0, e1, e2, e3, g0, g1, s0, s1, acc):
        esem = (e0, e1, e2, e3)
        gsem = (g0, g1)
        ssem = (s0, s1)
        c = lax.axis_index("c")
        s = lax.axis_index("s")
        base = s * rows_per_tile

        @pl.loop(0, 2 * CH)
        def _zero_row(r):
            for j in range(d // 32):
                frows_v[r, pl.ds(j * 32, 32)] = jnp.zeros((32,),
                                                          jnp.bfloat16)

        for m in (2, 3):
            for j in range(CH // LN):
                dst_r[m, pl.ds(j * LN, LN)] = jnp.zeros((LN,), jnp.int32)

        n_full, rem = divmod(rows_per_tile, CH)
        for k in range(n_full):
            pltpu.sync_copy(frows_v.at[pl.ds(0, CH)],
                            acc.at[pl.ds(base + k * CH, CH)])
        if rem:
            pltpu.sync_copy(frows_v.at[pl.ds(0, rem)],
                            acc.at[pl.ds(base + n_full * CH, rem)])

        plsc.subcore_barrier()

        def gbuf(b):
            return grows_v.at[pl.ds(b * CH, CH)]

        def fbuf(b):
            return frows_v.at[pl.ds(b * CH, CH)]

        def e_start(m, ci):
            pltpu.async_copy(srcs.at[c, s, ci], src_r.at[m], esem[m])
            pltpu.async_copy(dsts.at[c, s, ci], dst_r.at[m], esem[m])
            pltpu.async_copy(ws.at[c, s, ci], w_r.at[m], esem[m])

        def e_wait(m, ci):
            pltpu.make_async_copy(srcs.at[c, s, ci], src_r.at[m],
                                  esem[m]).wait()
            pltpu.make_async_copy(dsts.at[c, s, ci], dst_r.at[m],
                                  esem[m]).wait()
            pltpu.make_async_copy(ws.at[c, s, ci], w_r.at[m],
                                  esem[m]).wait()

        def g_start(b, m):
            pltpu.async_copy(ybf.at[src_r.at[m]], gbuf(b), gsem[b])

        def g_wait(b, m):
            pltpu.make_async_copy(ybf.at[src_r.at[m]], gbuf(b),
                                  gsem[b]).wait()

        def s_start(b, m):
            pltpu.async_copy(fbuf(b), acc.at[dst_r.at[m]], ssem[b], add=True)

        def s_wait(b, m):
            pltpu.make_async_copy(fbuf(b), acc.at[dst_r.at[m]],
                                  ssem[b]).wait()

        def scale(b, m):
            @pl.loop(0, CH // LN)
            def _grp(g):
                wv = w_r[m, pl.ds(g * LN, LN)]
                for i in range(LN):
                    sp = _lane_splat(wv, i)
                    spb = plsc.pack(sp, sp,
                                    format=plsc.PackFormat.INTERLEAVED)
                    r = b * CH + g * LN + i
                    for g2 in range(d // 32):
                        slc = pl.ds(g2 * 32, 32)
                        frows_v[r, slc] = grows_v[r, slc] * spb

        s_start(0, 2)
        s_start(1, 3)
        e_start(0, 0)
        e_start(1, 1)
        e_wait(0, 0)
        g_start(0, 0)

        n_iter = chunks // 4

        @pl.loop(0, n_iter)
        def _ring(h):
            c0 = 4 * h
            for k in range(4):
                ci = c0 + k
                b = k % 2
                mn = (k + 1) % 4
                s_wait(b, k)
                if k < 2:
                    e_start(k + 2, ci + 2)
                else:
                    @pl.when(h < n_iter - 1)
                    def _epf():
                        e_start((k + 2) % 4, ci + 2)
                if k < 3:
                    e_wait(mn, ci + 1)
                    g_start(1 - b, mn)
                else:
                    @pl.when(h < n_iter - 1)
                    def _gpf():
                        e_wait(mn, ci + 1)
                        g_start(1 - b, mn)
                g_wait(b, k)
                scale(b, k)
                s_start(b, k)

        s_wait(0, 2)
        s_wait(1, 3)

        plsc.subcore_barrier()
        pltpu.sync_copy(acc.at[pl.ds(base, rows_per_tile)],
                        out.at[c, pl.ds(base, rows_per_tile)])

    return sc_fn


def kernel(Y, X, edge_weight, deg, alp, lam, edge_index):
    n, d = Y.shape
    e = edge_weight.shape[0]
    chunks = 4 * (-(-e // (NC * NS * CH * 4)))
    epad = NC * NS * chunks * CH
    n2 = NS * 8 * (-(-n // (NS * 8)))

    src = edge_index[0].astype(jnp.int32)
    dst = edge_index[1].astype(jnp.int32)
    w = edge_weight.astype(jnp.float32)
    pad = epad - e
    if pad:
        src = jnp.concatenate([src, jnp.zeros((pad,), jnp.int32)])
        dst = jnp.concatenate([dst, jnp.zeros((pad,), jnp.int32)])
        w = jnp.concatenate([w, jnp.zeros((pad,), jnp.float32)])
    src4 = src.reshape(NC, NS, chunks, CH)
    dst4 = dst.reshape(NC, NS, chunks, CH)
    w4 = w.reshape(NC, NS, chunks, CH)
    ypad = Y
    deg_pad = deg
    if n2 > n:
        ypad = jnp.concatenate([Y, jnp.zeros((n2 - n, d), jnp.float32)])
        deg_pad = jnp.concatenate([deg, jnp.ones((n2 - n,), jnp.float32)])
    lam11 = lam.reshape(1, 1)
    alp11 = alp.reshape(1, 1)

    ybf = pl.pallas_call(
        _scale_y_body,
        out_shape=jax.ShapeDtypeStruct((n2, d), jnp.bfloat16),
    )(ypad, deg_pad[:, None], lam11)

    partials = _make_sc_kernel(n2, d, chunks)(ybf, src4, dst4, w4)[:, :n, :]

    blk = 2000
    out = pl.pallas_call(
        _combine_body,
        grid=(n // blk,),
        in_specs=[
            pl.BlockSpec((blk, d), lambda i: (i, 0)),
            pl.BlockSpec((blk, d), lambda i: (i, 0)),
            pl.BlockSpec((blk, 1), lambda i: (i, 0)),
            pl.BlockSpec((NC, blk, d), lambda i: (0, i, 0)),
            pl.BlockSpec((1, 1), lambda i: (0, 0)),
            pl.BlockSpec((1, 1), lambda i: (0, 0)),
        ],
        out_specs=pl.BlockSpec((blk, d), lambda i: (i, 0)),
        out_shape=jax.ShapeDtypeStruct((n, d), jnp.float32),
    )(Y, X, deg[:, None], partials, alp11, lam11)
    return out


# speedup vs baseline: 1.3321x; 1.3001x over previous
import dataclasses
import functools

import jax
import jax.numpy as jnp
from jax import lax
from jax.experimental import pallas as pl
from jax.experimental.pallas import tpu as pltpu
from jax.experimental.pallas import tpu_sc as plsc

NC = 2
NS = 16
LN = 16
CH = 128

_PERM32 = [0, 16, 1, 17, 2, 18, 3, 19, 4, 20, 5, 21, 6, 22, 7, 23,
           8, 24, 9, 25, 10, 26, 11, 27, 12, 28, 13, 29, 14, 30, 15, 31]


def _lane_splat(vec, i):
    idx = jnp.full((LN, 1), i, jnp.int32)
    dn = lax.GatherDimensionNumbers(
        offset_dims=(), collapsed_slice_dims=(0,), start_index_map=(0,))
    return lax.gather(vec, idx, dn, slice_sizes=(1,),
                      mode=lax.GatherScatterMode.PROMISE_IN_BOUNDS)


def _scale_y_body(y_ref, deg_ref, lam_ref, h_ref):
    lam = lam_ref[0, 0]
    dl = lam * deg_ref[...] + (1.0 - lam)
    yp = (y_ref[...] * lax.rsqrt(dl)).astype(jnp.bfloat16)
    dh = y_ref.shape[1] // 2
    h_ref[0] = yp[:, :dh]
    h_ref[1] = yp[:, dh:]


def _combine_body(y_ref, x_ref, deg_ref, h_ref, alp_ref, lam_ref, o_ref):
    alp = alp_ref[0, 0]
    lam = lam_ref[0, 0]
    dl = lam * deg_ref[...] + (1.0 - lam)
    nh = lax.rsqrt(dl)
    agg = jnp.concatenate([h_ref[0], h_ref[1]], axis=1)
    o_ref[...] = ((1.0 - alp) * y_ref[...]
                  + (alp * lam) * (nh * agg)
                  + alp * (x_ref[...] / dl))


def _make_sc_kernel(n2, dh, chunks):
    rows_per_tile = n2 // NS
    mesh = plsc.VectorSubcoreMesh(core_axis_name="c", subcore_axis_name="s")
    cp = pltpu.CompilerParams()
    for field, val in (("needs_layout_passes", False),
                       ("use_tc_tiling_on_sc", False)):
        if field in pltpu.CompilerParams.__dataclass_fields__:
            cp = dataclasses.replace(cp, **{field: val})

    @functools.partial(
        pl.kernel,
        mesh=mesh,
        compiler_params=cp,
        out_type=jax.ShapeDtypeStruct((NC, n2, dh), jnp.float32),
        scratch_types=[
            pltpu.VMEM((chunks, CH), jnp.int32),
            pltpu.VMEM((chunks, CH), jnp.int32),
            pltpu.VMEM((chunks, CH), jnp.float32),
            pltpu.VMEM((2 * CH, dh), jnp.bfloat16),
            pltpu.VMEM((2 * CH, dh), jnp.float32),
            pltpu.SemaphoreType.DMA,
            pltpu.SemaphoreType.DMA,
            pltpu.SemaphoreType.DMA,
            pltpu.SemaphoreType.DMA,
            pltpu.VMEM_SHARED((n2, dh), jnp.float32),
        ],
    )
    def sc_fn(yh, srcs, dsts, ws, out,
              src_v, dst_v, w_v, grows_v, frows_v,
              g0, g1, s0, s1, acc):
        gsem = (g0, g1)
        ssem = (s0, s1)
        c = lax.axis_index("c")
        s = lax.axis_index("s")
        base = s * rows_per_tile

        pltpu.sync_copy(srcs.at[s], src_v)
        pltpu.sync_copy(dsts.at[s], dst_v)
        pltpu.sync_copy(ws.at[s], w_v)

        @pl.loop(0, 2 * CH)
        def _zero_row(r):
            for j in range(dh // LN):
                frows_v[r, pl.ds(j * LN, LN)] = jnp.zeros((LN,), jnp.float32)

        n_full, rem = divmod(rows_per_tile, CH)
        for k in range(n_full):
            pltpu.sync_copy(frows_v.at[pl.ds(0, CH)],
                            acc.at[pl.ds(base + k * CH, CH)])
        if rem:
            pltpu.sync_copy(frows_v.at[pl.ds(0, rem)],
                            acc.at[pl.ds(base + n_full * CH, rem)])

        plsc.subcore_barrier()

        def gbuf(b):
            return grows_v.at[pl.ds(b * CH, CH)]

        def fbuf(b):
            return frows_v.at[pl.ds(b * CH, CH)]

        def g_start(b, ci):
            pltpu.async_copy(yh.at[c].at[src_v.at[ci]], gbuf(b), gsem[b])

        def g_wait(b, ci):
            pltpu.make_async_copy(yh.at[c].at[src_v.at[ci]], gbuf(b),
                                  gsem[b]).wait()

        def s_start(b, ci):
            pltpu.async_copy(fbuf(b), acc.at[dst_v.at[ci]], ssem[b], add=True)

        def s_wait(b, ci):
            pltpu.make_async_copy(fbuf(b), acc.at[dst_v.at[ci]],
                                  ssem[b]).wait()

        def scale(b, ci):
            for g in range(CH // LN):
                wv = w_v[ci, pl.ds(g * LN, LN)]
                for i in range(LN):
                    sp = _lane_splat(wv, i)
                    e = g * LN + i
                    for g2 in range(dh // 32):
                        packed = grows_v[b * CH + e, pl.ds(g2 * 32, 32)]
                        lo, hi = plsc.unpack(
                            packed, format=plsc.PackFormat.INTERLEAVED)
                        frows_v[b * CH + e, pl.ds(g2 * 32, LN)] = lo * sp
                        frows_v[b * CH + e, pl.ds(g2 * 32 + LN, LN)] = hi * sp

        g_start(0, 0)
        s_start(0, 0)
        s_start(1, 0)

        n_iter = chunks // 2

        @pl.loop(0, n_iter)
        def _ring(h):
            c0 = 2 * h
            for b in range(2):
                ci = c0 + b
                bn = 1 - b
                g_wait(b, ci)
                if b == 0:
                    g_start(bn, ci + 1)
                else:
                    @pl.when(h < n_iter - 1)
                    def _prefetch():
                        g_start(bn, ci + 1)
                s_wait(b, ci)
                scale(b, ci)
                s_start(b, ci)

        s_wait(0, chunks - 2)
        s_wait(1, chunks - 1)

        plsc.subcore_barrier()
        pltpu.sync_copy(acc.at[pl.ds(base, rows_per_tile)],
                        out.at[c, pl.ds(base, rows_per_tile)])

    return sc_fn


def kernel(Y, X, edge_weight, deg, alp, lam, edge_index):
    n, d = Y.shape
    e = edge_weight.shape[0]
    dh = d // 2
    chunks = 2 * (-(-e // (NS * CH * 2)))
    epad = NS * chunks * CH
    n2 = NS * 8 * (-(-n // (NS * 8)))

    src = edge_index[0].astype(jnp.int32)
    dst = edge_index[1].astype(jnp.int32)
    w = edge_weight.astype(jnp.float32)
    pad = epad - e
    if pad:
        src = jnp.concatenate([src, jnp.zeros((pad,), jnp.int32)])
        dst = jnp.concatenate([dst, jnp.zeros((pad,), jnp.int32)])
        w = jnp.concatenate([w, jnp.zeros((pad,), jnp.float32)])
    src3 = src.reshape(NS, chunks, CH)
    dst3 = dst.reshape(NS, chunks, CH)
    w3 = w.reshape(NS, chunks, CH)
    ypad = Y
    deg_pad = deg
    if n2 > n:
        ypad = jnp.concatenate([Y, jnp.zeros((n2 - n, d), jnp.float32)])
        deg_pad = jnp.concatenate([deg, jnp.ones((n2 - n,), jnp.float32)])
    lam11 = lam.reshape(1, 1)
    alp11 = alp.reshape(1, 1)

    yh = pl.pallas_call(
        _scale_y_body,
        out_shape=jax.ShapeDtypeStruct((NC, n2, dh), jnp.bfloat16),
    )(ypad, deg_pad[:, None], lam11)
    if _PERM32 != list(range(32)):
        perm = jnp.asarray([g * 32 + p for g in range(dh // 32)
                            for p in _PERM32], dtype=jnp.int32)
        yh = yh[:, :, perm]

    halves = _make_sc_kernel(n2, dh, chunks)(yh, src3, dst3, w3)[:, :n, :]

    blk = 2000
    out = pl.pallas_call(
        _combine_body,
        grid=(n // blk,),
        in_specs=[
            pl.BlockSpec((blk, d), lambda i: (i, 0)),
            pl.BlockSpec((blk, d), lambda i: (i, 0)),
            pl.BlockSpec((blk, 1), lambda i: (i, 0)),
            pl.BlockSpec((NC, blk, dh), lambda i: (0, i, 0)),
            pl.BlockSpec((1, 1), lambda i: (0, 0)),
            pl.BlockSpec((1, 1), lambda i: (0, 0)),
        ],
        out_specs=pl.BlockSpec((blk, d), lambda i: (i, 0)),
        out_shape=jax.ShapeDtypeStruct((n, d), jnp.float32),
    )(Y, X, deg[:, None], halves, alp11, lam11)
    return out
